# Initial kernel scaffold; baseline (speedup 1.0000x reference)
import jax
import jax.numpy as jnp
from jax.experimental import pallas as pl


def kernel(x_sacc, x_full, params):
    raise NotImplementedError("write your pallas kernel here")


# speedup vs baseline: 6.5075x; 6.5075x over previous
import functools
import math

import jax
import jax.numpy as jnp
from jax import lax
from jax.experimental import pallas as pl
from jax.experimental.pallas import tpu as pltpu

D = 1024
BD = 2048
H = 16
DH = 64
BLK = 64
WS = 128
NSACC = 2
TOPK = 16


def _f32(x):
    return jnp.asarray(x, jnp.float32)


def _dot(a, b):
    return jnp.dot(a, b, preferred_element_type=jnp.float32)


def _ln_rows(x, g, b, eps=1e-5):
    m = jnp.mean(x, axis=-1, keepdims=True)
    v = jnp.mean((x - m) ** 2, axis=-1, keepdims=True)
    return (x - m) * lax.rsqrt(v + eps) * g + b


def _gelu(x):
    return 0.5 * x * (1.0 + lax.erf(x * (2.0 ** -0.5)))


def _periph_stage1_kernel(x_ref, wc_ref, bc_ref, xmid_ref, std_ref, max_ref):
    x = x_ref[0]
    xmid_ref[0] = _dot(x, wc_ref[...].T) + bc_ref[...]
    mean = jnp.mean(x, axis=0, keepdims=True)
    var = jnp.sum((x - mean) ** 2, axis=0, keepdims=True) / (BLK - 1)
    std_ref[0] = jnp.sqrt(var)
    max_ref[0] = jnp.max(x, axis=0, keepdims=True)


def _periph_stage2_kernel(cin_ref, wf_ref, cb_ref, sv_ref, sw_ref, sb_ref,
                          mv_ref, mw_ref, mb_ref, pc_ref, ps_ref, pm_ref,
                          pb_ref, g_ref, b_ref, pos_ref, pmap_ref, state_ref):
    co = _dot(cin_ref[...], wf_ref[...]) + cb_ref[...]
    so = _dot(sv_ref[...], sw_ref[...].T) + sb_ref[...]
    mo = _dot(mv_ref[...], mw_ref[...].T) + mb_ref[...]
    pre = (_dot(co, pc_ref[...]) + _dot(so, ps_ref[...]) + _dot(mo, pm_ref[...])
           + pb_ref[...])
    pmap = _ln_rows(pre, g_ref[...], b_ref[...]) + pos_ref[...]
    pmap_ref[...] = pmap
    nb = pmap.shape[0] // state_ref.shape[0]
    for bi in range(state_ref.shape[0]):
        state_ref[bi:bi + 1, :] = jnp.mean(pmap[bi * nb:(bi + 1) * nb], axis=0,
                                           keepdims=True)


def _hkv_kernel(x_ref, l1g_ref, l1b_ref, n1g_ref, n1b_ref, wk_ref, bk_ref,
                wv_ref, bv_ref, h_ref, kt_ref, vt_ref):
    h = _ln_rows(x_ref[...], l1g_ref[...], l1b_ref[...])
    h_ref[...] = h
    g = _ln_rows(h, n1g_ref[...], n1b_ref[...])
    kt_ref[...] = _dot(g, wk_ref[...].T) + bk_ref[...]
    vt_ref[...] = _dot(g, wv_ref[...].T) + bv_ref[...]


def _controller_kernel(state_ref, pmap_ref, cqw_ref, cqb_ref, ckw_ref,
                       ckb_ref, n1g_ref, n1b_ref, wq_ref, bq_ref, wk_ref,
                       bk_ref, wv_ref, bv_ref,
                       scores_ref, ti_ref, tw_ref, qf_ref, ks_ref, vs_ref):
    B = state_ref.shape[0]
    nb = pmap_ref.shape[1]
    state = state_ref[...]
    pm = pmap_ref[...].reshape(B * nb, D)
    q = _dot(state, cqw_ref[...].T) + cqb_ref[...]
    kk = _dot(pm, ckw_ref[...].T) + ckb_ref[...]
    sfull = _dot(q, kk.T) / math.sqrt(D)
    rows = [sfull[bi:bi + 1, bi * nb:(bi + 1) * nb] for bi in range(B)]
    scores = jnp.concatenate(rows, axis=0)
    scores_ref[...] = scores

    iota = lax.broadcasted_iota(jnp.int32, (B, nb), 1)
    work = scores
    tvs, tis = [], []
    for _ in range(TOPK):
        m = jnp.max(work, axis=1, keepdims=True)
        idx = jnp.min(jnp.where(work == m, iota, nb), axis=1, keepdims=True)
        tvs.append(m)
        tis.append(idx)
        work = jnp.where(iota == idx, -jnp.inf, work)
    tv = jnp.concatenate(tvs, axis=1)
    ti = jnp.concatenate(tis, axis=1)
    ti_ref[...] = ti
    e = jnp.exp((tv - tv[:, 0:1]) / 5.0)
    tw_ref[...] = e / jnp.sum(e, axis=1, keepdims=True)

    g = _ln_rows(state, n1g_ref[...], n1b_ref[...])
    qf_ref[...] = _dot(g, wq_ref[...].T) + bq_ref[...]
    ks_ref[...] = _dot(g, wk_ref[...].T) + bk_ref[...]
    vs_ref[...] = _dot(g, wv_ref[...].T) + bv_ref[...]


def _foveal_attn_kernel(starts_ref, astarts_ref, kt_ref, vt_ref, qm_ref,
                        ks_ref, vs_ref, ex_ref, ctxv_ref, *, nacc):
    qm = qm_ref[...]
    ex = ex_ref[...]
    s_state = _dot(ks_ref[...], qm) / 8.0
    vs = vs_ref[...]

    saccs, vaccs = [], []
    for j in range(nacc):
        a0 = astarts_ref[0, j] * 8
        kacc = kt_ref[pl.ds(a0, WS), :]
        vacc = vt_ref[pl.ds(a0, WS), :]
        saccs.append(_dot(kacc, qm) / 8.0)
        vaccs.append(vacc)

    for k in range(TOPK):
        st = starts_ref[0, k] * 8
        kwin = kt_ref[pl.ds(st, WS), :]
        vwin = vt_ref[pl.ds(st, WS), :]
        sw = _dot(kwin, qm) / 8.0
        m = jnp.maximum(jnp.max(sw, axis=0, keepdims=True), s_state)
        for sa in saccs:
            m = jnp.maximum(m, jnp.max(sa, axis=0, keepdims=True))
        ew = jnp.exp(sw - m)
        es = jnp.exp(s_state - m)
        denom = jnp.sum(ew, axis=0, keepdims=True) + es
        eas = []
        for sa in saccs:
            ea = jnp.exp(sa - m)
            eas.append(ea)
            denom = denom + jnp.sum(ea, axis=0, keepdims=True)
        inv = 1.0 / denom
        ctxv = jnp.sum(vwin * _dot(ew * inv, ex), axis=0, keepdims=True)
        ctxv = ctxv + vs * _dot(es * inv, ex)
        for ea, vacc in zip(eas, vaccs):
            ctxv = ctxv + jnp.sum(vacc * _dot(ea * inv, ex), axis=0,
                                  keepdims=True)
        ctxv_ref[k:k + 1, :] = ctxv


def _foveal_dense_kernel(ctxv_ref, st32_ref, tw_ref, ow_ref, ob_ref, n2g_ref,
                         n2b_ref, w1_ref, b1_ref, w2_ref, b2_ref, wst_ref):
    B, K = tw_ref.shape
    s = st32_ref[...] + _dot(ctxv_ref[...], ow_ref[...].T) + ob_ref[...]
    u = _ln_rows(s, n2g_ref[...], n2b_ref[...])
    m1 = _gelu(_dot(u, w1_ref[...].T) + b1_ref[...])
    s2 = s + _dot(m1, w2_ref[...].T) + b2_ref[...]
    for bi in range(B):
        wst_ref[bi:bi + 1, :] = _dot(tw_ref[bi:bi + 1, :],
                                     s2[bi * K:(bi + 1) * K, :])


def _memory_attn_kernel(astarts_ref, alpha_ref, pmap_ref, h_ref, ng_ref,
                        nb_ref, wq_ref, bq_ref, wk_ref, bk_ref, wv_ref,
                        bv_ref, wo_ref, bo_ref, out_ref, *, nacc):
    B = pmap_ref.shape[0]
    alpha = alpha_ref[0, 0]
    for bi in range(B):
        pm = pmap_ref[bi]
        pn = _ln_rows(pm, ng_ref[...], nb_ref[...])
        q = _dot(pn, wq_ref[...].T) + bq_ref[...]
        rows = [h_ref[bi, pl.ds(astarts_ref[bi, j] * 8, WS), :]
                for j in range(nacc)]
        a = jnp.concatenate(rows, axis=0) if nacc > 1 else rows[0]
        ka = _dot(a, wk_ref[...].T) + bk_ref[...]
        va = _dot(a, wv_ref[...].T) + bv_ref[...]
        pieces = []
        for hh in range(H):
            sl = slice(hh * DH, (hh + 1) * DH)
            sc = _dot(q[:, sl], ka[:, sl].T) / 8.0
            sc = sc - jnp.max(sc, axis=1, keepdims=True)
            p = jnp.exp(sc)
            p = p / jnp.sum(p, axis=1, keepdims=True)
            pieces.append(_dot(p, va[:, sl]))
        ctx = jnp.concatenate(pieces, axis=1)
        delta = _dot(ctx, wo_ref[...].T) + bo_ref[...]
        out_ref[bi] = pm + alpha * delta


def _final_kernel(res_ref, state_ref, og_ref, obn_ref, ow_ref, ob_ref,
                  l2g_ref, l2b_ref, w1_ref, b1_ref, w2_ref, b2_ref, out_ref):
    srow = _ln_rows(state_ref[0], og_ref[...], obn_ref[...])
    orow = _dot(srow, ow_ref[...].T) + ob_ref[...]
    x = res_ref[0] + orow
    u = _ln_rows(x, l2g_ref[...], l2b_ref[...])
    m1 = _gelu(_dot(u, w1_ref[...].T) + b1_ref[...])
    out_ref[0] = x + _dot(m1, w2_ref[...].T) + b2_ref[...]


def kernel(x_sacc, x_full, params):
    p = params
    B, N, _ = x_sacc.shape
    nb = N // BLK
    r1 = lambda v: v.reshape(1, -1)

    xf_blocks = x_full.reshape(B * nb, BLK, BD)
    xmid, stdv, maxv = pl.pallas_call(
        _periph_stage1_kernel,
        grid=(B * nb,),
        in_specs=[
            pl.BlockSpec((1, BLK, BD), lambda i: (i, 0, 0)),
            pl.BlockSpec((256, BD), lambda i: (0, 0)),
            pl.BlockSpec((1, 256), lambda i: (0, 0)),
        ],
        out_specs=[
            pl.BlockSpec((1, BLK, 256), lambda i: (i, 0, 0)),
            pl.BlockSpec((1, 1, BD), lambda i: (i, 0, 0)),
            pl.BlockSpec((1, 1, BD), lambda i: (i, 0, 0)),
        ],
        out_shape=[
            jax.ShapeDtypeStruct((B * nb, BLK, 256), jnp.float32),
            jax.ShapeDtypeStruct((B * nb, 1, BD), jnp.float32),
            jax.ShapeDtypeStruct((B * nb, 1, BD), jnp.float32),
        ],
    )(xf_blocks, p['p_conv_proj_w'], r1(p['p_conv_proj_b']))

    conv_in = xmid.reshape(B * nb, BLK * 256)
    wflat = p['p_conv_w'].transpose(2, 1, 0).reshape(BLK * 256, 256)
    pos = jnp.tile(p['p_pos'][:nb], (B, 1))
    pmap_flat, state = pl.pallas_call(
        _periph_stage2_kernel,
        out_shape=[
            jax.ShapeDtypeStruct((B * nb, D), jnp.float32),
            jax.ShapeDtypeStruct((B, D), jnp.float32),
        ],
    )(conv_in, wflat, r1(p['p_conv_b']), stdv.reshape(B * nb, BD),
      p['p_std_w'], r1(p['p_std_b']), maxv.reshape(B * nb, BD),
      p['p_max_w'], r1(p['p_max_b']),
      p['p_proj_w'][:, :256].T, p['p_proj_w'][:, 256:512].T,
      p['p_proj_w'][:, 512:].T, r1(p['p_proj_b']),
      r1(p['p_norm_g']), r1(p['p_norm_b']), pos)
    pmap = pmap_flat.reshape(B, nb, D)

    fw, fb = p['f_in_w'], p['f_in_b']
    bm = 512
    x_rows = x_sacc.reshape(B * N, D)
    h_rows, ktab, vtab = pl.pallas_call(
        _hkv_kernel,
        grid=(B * N // bm,),
        in_specs=[pl.BlockSpec((bm, D), lambda i: (i, 0))] +
                 [pl.BlockSpec((1, D), lambda i: (0, 0))] * 4 +
                 [pl.BlockSpec((D, D), lambda i: (0, 0)),
                  pl.BlockSpec((1, D), lambda i: (0, 0)),
                  pl.BlockSpec((D, D), lambda i: (0, 0)),
                  pl.BlockSpec((1, D), lambda i: (0, 0))],
        out_specs=[pl.BlockSpec((bm, D), lambda i: (i, 0))] * 3,
        out_shape=[jax.ShapeDtypeStruct((B * N, D), jnp.float32)] * 3,
    )(x_rows, r1(p['ln1_g']), r1(p['ln1_b']), r1(p['f_n1_g']),
      r1(p['f_n1_b']), fw[D:2 * D], r1(fb[D:2 * D]), fw[2 * D:],
      r1(fb[2 * D:]))
    h3 = h_rows.reshape(B, N, D)
    kt3 = ktab.reshape(B, N, D)
    vt3 = vtab.reshape(B, N, D)

    head_mask = (lax.broadcasted_iota(jnp.int32, (D, H), 0) // DH ==
                 lax.broadcasted_iota(jnp.int32, (D, H), 1)).astype(jnp.float32)
    expander = head_mask.T

    controller = pl.pallas_call(
        _controller_kernel,
        out_shape=[
            jax.ShapeDtypeStruct((B, nb), jnp.float32),
            jax.ShapeDtypeStruct((B, TOPK), jnp.int32),
            jax.ShapeDtypeStruct((B, TOPK), jnp.float32),
            jax.ShapeDtypeStruct((B, D), jnp.float32),
            jax.ShapeDtypeStruct((B, D), jnp.float32),
            jax.ShapeDtypeStruct((B, D), jnp.float32),
        ],
    )

    fps, flogits = [], []
    acc_starts = []
    for t in range(NSACC):
        scores, ti, tw, qf, ks, vs = controller(
            state, pmap, p['c_q_w'], r1(p['c_q_b']), p['c_k_w'],
            r1(p['c_k_b']), r1(p['f_n1_g']), r1(p['f_n1_b']),
            fw[:D], r1(fb[:D]), fw[D:2 * D], r1(fb[D:2 * D]),
            fw[2 * D:], r1(fb[2 * D:]))
        fps.append(ti[:, 0] * BLK)
        flogits.append(scores)
        starts = jnp.clip(ti * BLK - WS // 2, 0, N - WS) // 8

        qmat = qf[:, :, None] * head_mask[None]
        astack = (jnp.stack(acc_starts, axis=1) if acc_starts
                  else jnp.zeros((B, 1), jnp.int32))
        nacc = len(acc_starts)

        ctxv_parts = []
        fov = pl.pallas_call(
            functools.partial(_foveal_attn_kernel, nacc=nacc),
            in_specs=[pl.BlockSpec(memory_space=pltpu.SMEM),
                      pl.BlockSpec(memory_space=pltpu.SMEM)] +
                     [pl.BlockSpec()] * 6,
            out_shape=jax.ShapeDtypeStruct((TOPK, D), jnp.float32),
        )
        for bi in range(B):
            ctxv_parts.append(fov(
                starts[bi:bi + 1], astack[bi:bi + 1], kt3[bi], vt3[bi],
                qmat[bi], ks[bi:bi + 1], vs[bi:bi + 1], expander))
        ctxv32 = jnp.concatenate(ctxv_parts, axis=0)

        st32 = jnp.repeat(state, TOPK, axis=0)
        state = pl.pallas_call(
            _foveal_dense_kernel,
            out_shape=jax.ShapeDtypeStruct((B, D), jnp.float32),
        )(ctxv32, st32, tw, p['f_out_w'], r1(p['f_out_b']),
          r1(p['f_n2_g']), r1(p['f_n2_b']), p['f_ffn1_w'],
          r1(p['f_ffn1_b']), p['f_ffn2_w'], r1(p['f_ffn2_b']))

        acc_starts.append(starts[:, 0])
        astack2 = jnp.stack(acc_starts, axis=1)

        tt = jnp.array([[t / NSACC]], dtype=jnp.float32)
        a1 = _gelu(tt @ p['g1_w'].T + p['g1_b'])
        alpha = jax.nn.sigmoid(a1 @ p['g2_w'].T + p['g2_b'])

        mw, mb = p['m_in_w'], p['m_in_b']
        pmap = pl.pallas_call(
            functools.partial(_memory_attn_kernel, nacc=t + 1),
            in_specs=[pl.BlockSpec(memory_space=pltpu.SMEM),
                      pl.BlockSpec(memory_space=pltpu.SMEM)] +
                     [pl.BlockSpec()] * 12,
            out_shape=jax.ShapeDtypeStruct((B, nb, D), jnp.float32),
        )(astack2, alpha, pmap, h3, r1(p['m_norm_g']), r1(p['m_norm_b']),
          mw[:D], r1(mb[:D]), mw[D:2 * D], r1(mb[D:2 * D]), mw[2 * D:],
          r1(mb[2 * D:]), p['m_out_w'], r1(p['m_out_b']))

    bm2 = 256
    out = pl.pallas_call(
        _final_kernel,
        grid=(B, N // bm2),
        in_specs=[
            pl.BlockSpec((1, bm2, D), lambda b, i: (b, i, 0)),
            pl.BlockSpec((1, 1, D), lambda b, i: (b, 0, 0)),
        ] + [pl.BlockSpec((1, D), lambda b, i: (0, 0))] * 2 + [
            pl.BlockSpec((D, D), lambda b, i: (0, 0)),
            pl.BlockSpec((1, D), lambda b, i: (0, 0)),
            pl.BlockSpec((1, D), lambda b, i: (0, 0)),
            pl.BlockSpec((1, D), lambda b, i: (0, 0)),
            pl.BlockSpec((4 * D, D), lambda b, i: (0, 0)),
            pl.BlockSpec((1, 4 * D), lambda b, i: (0, 0)),
            pl.BlockSpec((D, 4 * D), lambda b, i: (0, 0)),
            pl.BlockSpec((1, D), lambda b, i: (0, 0)),
        ],
        out_specs=pl.BlockSpec((1, bm2, D), lambda b, i: (b, i, 0)),
        out_shape=jax.ShapeDtypeStruct((B, N, D), jnp.float32),
    )(x_sacc, state.reshape(B, 1, D), r1(p['o_norm_g']), r1(p['o_norm_b']),
      p['o_w'], r1(p['o_b']), r1(p['ln2_g']), r1(p['ln2_b']),
      p['mlp1_w'], r1(p['mlp1_b']), p['mlp2_w'], r1(p['mlp2_b']))

    return out, jnp.stack(fps), jnp.stack(flogits)
